# baseline (speedup 1.0000x reference)
import jax
import jax.numpy as jnp
from jax.experimental import pallas as pl
from jax.experimental.pallas import tpu as pltpu

_N_TOKENS = 32768
_D = 768
_E = 8
_BLK = 1024
_NBLK = _N_TOKENS // _BLK
_NBUF = 12


def _router_body(x_hbm, wt_ref, w_out_ref, i_out_ref, buf, sems):
    i = pl.program_id(0)

    def _copy(blk, slot):
        return pltpu.make_async_copy(
            x_hbm.at[pl.ds(blk * _BLK, _BLK), :],
            buf.at[slot],
            sems.at[slot],
        )

    @pl.when(i == 0)
    def _prologue():
        for s in range(_NBUF):
            _copy(s, s).start()

    slot = jax.lax.rem(i, _NBUF)
    _copy(i, slot).wait()
    s = jnp.dot(buf[slot], wt_ref[...], preferred_element_type=jnp.float32)
    m = jnp.max(s, axis=1, keepdims=True)
    lane = jax.lax.broadcasted_iota(jnp.int32, s.shape, 1)
    idx = jnp.min(jnp.where(s == m, lane, _E), axis=1, keepdims=True)
    w_out_ref[...] = m
    i_out_ref[...] = idx

    @pl.when(i + _NBUF < _NBLK)
    def _next():
        _copy(i + _NBUF, slot).start()


def kernel(x, W):
    wt = W.T
    weights, indices = pl.pallas_call(
        _router_body,
        grid=(_NBLK,),
        in_specs=[
            pl.BlockSpec(memory_space=pl.ANY),
            pl.BlockSpec((_D, _E), lambda i: (0, 0)),
        ],
        out_specs=[
            pl.BlockSpec((_BLK, 1), lambda i: (i, 0)),
            pl.BlockSpec((_BLK, 1), lambda i: (i, 0)),
        ],
        out_shape=[
            jax.ShapeDtypeStruct((_N_TOKENS, 1), jnp.float32),
            jax.ShapeDtypeStruct((_N_TOKENS, 1), jnp.int32),
        ],
        scratch_shapes=[
            pltpu.VMEM((_NBUF, _BLK, _D), jnp.float32),
            pltpu.SemaphoreType.DMA((_NBUF,)),
        ],
    )(x, wt)
    return (weights, indices)
